# baseline (speedup 1.0000x reference)
import functools

import jax
import jax.numpy as jnp
from jax import lax
from jax.experimental import pallas as pl
from jax.experimental.pallas import tpu as pltpu
from jax.experimental.pallas import tpu_sc as plsc

VOCAB = 100000
D = 768
PAD_IDX = 100000
B, T = 4, 8192

NC, NS, L = 2, 16, 16
NW = NC * NS
N_ROWS = B * T
ROWS_PER_W = N_ROWS // NW
C = 64
N_CHUNKS = ROWS_PER_W // C


def _body(emb_hbm, pos_hbm, idx_hbm, out_hbm,
          idx_v, safe_v, emb_v, acc_v, sem):
    wid = lax.axis_index("s") * NC + lax.axis_index("c")
    base = wid * ROWS_PER_W
    t0 = base % T

    def chunk_body(g, _):
        row0 = base + g * C
        pltpu.sync_copy(idx_hbm.at[pl.ds(row0, C)], idx_v)

        def prep(i, _):
            v = idx_v[pl.ds(i * L, L)]
            safe_v[pl.ds(i * L, L)] = jnp.where(v == PAD_IDX, 0, v)
            return 0

        lax.fori_loop(0, C // L, prep, 0)

        gather = pltpu.async_copy(emb_hbm.at[safe_v], emb_v, sem)
        pltpu.sync_copy(pos_hbm.at[pl.ds(t0 + g * C, C)], acc_v)
        gather.wait()

        def group_body(gg, _):
            vg = idx_v[pl.ds(gg * L, L)]
            scale_vec = jnp.where(vg == PAD_IDX,
                                  jnp.float32(0.0), jnp.float32(1.0))

            def row_body(j, _):
                r = gg * L + j
                scale = lax.gather(
                    scale_vec, jnp.full((L, 1), j, jnp.int32),
                    lax.GatherDimensionNumbers(
                        offset_dims=(), collapsed_slice_dims=(0,),
                        start_index_map=(0,)),
                    slice_sizes=(1,),
                    mode=lax.GatherScatterMode.PROMISE_IN_BOUNDS)
                for c in range(D // L):
                    sl = pl.ds(c * L, L)
                    plsc.addupdate(acc_v.at[r, sl], emb_v[r, sl] * scale)
                return 0

            lax.fori_loop(0, L, row_body, 0)
            return 0

        lax.fori_loop(0, C // L, group_body, 0)
        pltpu.sync_copy(acc_v, out_hbm.at[pl.ds(row0, C)])
        return 0

    lax.fori_loop(0, N_CHUNKS, chunk_body, 0)


@jax.jit
def _embed(x_flat, emb_table, pos_table):
    mesh = plsc.VectorSubcoreMesh(core_axis_name="c", subcore_axis_name="s")
    k = functools.partial(
        pl.kernel, mesh=mesh,
        out_type=jax.ShapeDtypeStruct((N_ROWS, D), jnp.float32),
        scratch_types=[
            pltpu.VMEM((C,), jnp.int32),
            pltpu.VMEM((C,), jnp.int32),
            pltpu.VMEM((C, D), jnp.float32),
            pltpu.VMEM((C, D), jnp.float32),
            pltpu.SemaphoreType.DMA,
        ],
    )(_body)
    return k(emb_table, pos_table, x_flat)


def kernel(x, emb_table, pos_table):
    x_flat = x.reshape(-1).astype(jnp.int32)
    out = _embed(x_flat, emb_table, pos_table)
    return out.reshape(B, T, D)
